# Initial kernel scaffold; baseline (speedup 1.0000x reference)
import jax
import jax.numpy as jnp
from jax.experimental import pallas as pl


def kernel(pred_loc, pred_score, priors_data, gt_data):
    raise NotImplementedError("write your pallas kernel here")


# speedup vs baseline: 16.1979x; 16.1979x over previous
import jax
import jax.numpy as jnp
from jax import lax
from jax.experimental import pallas as pl

_C = 21
_B = 32
_P = 8732
_O = 12
_GB = 8
_GRID = _B // _GB
_V0, _V1 = 0.1, 0.2
_NEG_POS = 3


def _sl1(x):
    ax = jnp.abs(x)
    return jnp.where(ax < 1.0, 0.5 * x * x, ax - 0.5)


def _body(pl_ref, ps_ref, pr_ref, gt_ref, acc_ref, out_ref):
    step = pl.program_id(0)

    pcx = pr_ref[0:1, :]
    pcy = pr_ref[1:2, :]
    pw = pr_ref[2:3, :]
    ph = pr_ref[3:4, :]
    pxmin = pcx - pw * 0.5
    pymin = pcy - ph * 0.5
    pxmax = pcx + pw * 0.5
    pymax = pcy + ph * 0.5
    area_p = pw * ph

    lane = lax.broadcasted_iota(jnp.int32, (_GB, _P), 1)
    gt = gt_ref[...]

    bto = None
    bti = None
    bp = []
    tx1 = []
    ty1 = []
    tx2 = []
    ty2 = []
    tlab = []
    for o in range(_O):
        x1 = gt[:, 0, o][:, None]
        y1 = gt[:, 1, o][:, None]
        x2 = gt[:, 2, o][:, None]
        y2 = gt[:, 3, o][:, None]
        tx1.append(x1)
        ty1.append(y1)
        tx2.append(x2)
        ty2.append(y2)
        tlab.append(gt[:, 4, o][:, None])
        iw = jnp.maximum(jnp.minimum(x2, pxmax) - jnp.maximum(x1, pxmin), 0.0)
        ih = jnp.maximum(jnp.minimum(y2, pymax) - jnp.maximum(y1, pymin), 0.0)
        inter = iw * ih
        area_t = (x2 - x1) * (y2 - y1)
        iou = inter / (area_t + area_p - inter)
        mo = jnp.max(iou, axis=1, keepdims=True)
        bp.append(jnp.min(jnp.where(iou == mo, lane, _P), axis=1, keepdims=True))
        if o == 0:
            bto = iou
            bti = jnp.zeros((_GB, _P), jnp.int32)
        else:
            m = iou > bto
            bti = jnp.where(m, o, bti)
            bto = jnp.where(m, iou, bto)

    for o in range(_O):
        eq = lane == bp[o]
        bto = jnp.where(eq, 2.0, bto)
        bti = jnp.where(eq, o, bti)

    conf = jnp.zeros((_GB, _P), jnp.int32)
    mx1 = jnp.zeros((_GB, _P), jnp.float32)
    my1 = jnp.zeros((_GB, _P), jnp.float32)
    mx2 = jnp.zeros((_GB, _P), jnp.float32)
    my2 = jnp.zeros((_GB, _P), jnp.float32)
    for o in range(_O):
        s = bti == o
        conf = jnp.where(s, tlab[o].astype(jnp.int32) + 1, conf)
        mx1 = jnp.where(s, tx1[o], mx1)
        my1 = jnp.where(s, ty1[o], my1)
        mx2 = jnp.where(s, tx2[o], mx2)
        my2 = jnp.where(s, ty2[o], my2)
    conf = jnp.where(bto < 0.5, 0, conf)
    pos = conf > 0
    posf = pos.astype(jnp.float32)

    g_cx = ((mx1 + mx2) * 0.5 - pcx) / (_V0 * pw)
    g_cy = ((my1 + my2) * 0.5 - pcy) / (_V0 * ph)
    g_w = jnp.log((mx2 - mx1) / pw) / _V1
    g_h = jnp.log((my2 - my1) / ph) / _V1
    ll = (_sl1(pl_ref[:, 0, :] - g_cx) + _sl1(pl_ref[:, 1, :] - g_cy)
          + _sl1(pl_ref[:, 2, :] - g_w) + _sl1(pl_ref[:, 3, :] - g_h))
    loss_l = jnp.sum(ll * posf)

    mx = ps_ref[:, 0, :]
    for c in range(1, _C):
        mx = jnp.maximum(mx, ps_ref[:, c, :])
    se = jnp.zeros((_GB, _P), jnp.float32)
    picked = jnp.zeros((_GB, _P), jnp.float32)
    for c in range(_C):
        s_c = ps_ref[:, c, :]
        se = se + jnp.exp(s_c - mx)
        picked = jnp.where(conf == c, s_c, picked)
    ce = jnp.log(se) + mx - picked
    pos_ce = jnp.sum(ce * posf)
    num_pos = jnp.sum(posf, axis=1, keepdims=True)

    lc = jnp.maximum(jnp.where(pos, 0.0, ce), 0.0)
    bits = lax.bitcast_convert_type(lc, jnp.int32)
    k = jnp.minimum(num_pos.astype(jnp.int32) * _NEG_POS, _P - 1)

    def bs_body(_, carry):
        lo, hi = carry
        mid = lo + lax.div(hi - lo, 2)
        cnt = jnp.sum((bits > mid).astype(jnp.int32), axis=1, keepdims=True)
        ge = cnt >= k
        return jnp.where(ge, mid + 1, lo), jnp.where(ge, hi, mid)

    lo0 = jnp.zeros((_GB, 1), jnp.int32)
    hi0 = jnp.full((_GB, 1), 2**31 - 1, jnp.int32)
    _, hi = lax.fori_loop(0, 32, bs_body, (lo0, hi0))
    thr = lax.bitcast_convert_type(hi, jnp.float32)
    gtm = lc > thr
    cgt = jnp.sum(gtm.astype(jnp.int32), axis=1, keepdims=True)
    gtsum = jnp.sum(jnp.where(gtm, lc, 0.0), axis=1, keepdims=True)
    rem = (k - cgt).astype(jnp.float32)
    neg_sum = jnp.sum(gtsum + jnp.where(k > cgt, rem * thr, 0.0))

    vec = jnp.concatenate(
        [loss_l[None, None], pos_ce[None, None],
         jnp.sum(num_pos)[None, None], neg_sum[None, None]], axis=1)

    @pl.when(step == 0)
    def _():
        acc_ref[...] = jnp.zeros((1, 4), jnp.float32)

    acc_ref[...] += vec

    @pl.when(step == _GRID - 1)
    def _():
        a = acc_ref[...]
        n = a[0:1, 2:3]
        out_ref[...] = jnp.concatenate(
            [a[0:1, 0:1] / n, (a[0:1, 1:2] + a[0:1, 3:4]) / n], axis=1)


def kernel(pred_loc, pred_score, priors_data, gt_data):
    pl_t = jnp.transpose(pred_loc, (0, 2, 1))
    ps_t = jnp.transpose(pred_score, (0, 2, 1))
    pr_t = priors_data.T
    gt_t = jnp.transpose(gt_data, (0, 2, 1))
    _, out = pl.pallas_call(
        _body,
        grid=(_GRID,),
        in_specs=[
            pl.BlockSpec((_GB, 4, _P), lambda i: (i, 0, 0)),
            pl.BlockSpec((_GB, _C, _P), lambda i: (i, 0, 0)),
            pl.BlockSpec((4, _P), lambda i: (0, 0)),
            pl.BlockSpec((_GB, 5, _O), lambda i: (i, 0, 0)),
        ],
        out_specs=[
            pl.BlockSpec((1, 4), lambda i: (0, 0)),
            pl.BlockSpec((1, 2), lambda i: (0, 0)),
        ],
        out_shape=[
            jax.ShapeDtypeStruct((1, 4), jnp.float32),
            jax.ShapeDtypeStruct((1, 2), jnp.float32),
        ],
    )(pl_t, ps_t, pr_t, gt_t)
    return (out[0, 0], out[0, 1])
